# Initial kernel scaffold; baseline (speedup 1.0000x reference)
import jax
import jax.numpy as jnp
from jax.experimental import pallas as pl


def kernel(xyz1, xyz2, points1, points2, W0, b0, gamma0, beta0, W1, b1, gamma1, beta1):
    raise NotImplementedError("write your pallas kernel here")


# speedup vs baseline: 17.5876x; 17.5876x over previous
import functools

import jax
import jax.numpy as jnp
from jax import lax
from jax.experimental import pallas as pl


NT1 = 256
NT2 = 512


def _k1_body(x1_ref, x2t_ref, p1_ref, p2_ref, w0a_ref, w0b_ref, b0_ref,
             h0_ref, s0_ref, q0_ref):
    S = x2t_ref.shape[2]
    x1 = x1_ref[0]
    x2t = x2t_ref[0]
    x1c = [x1[:, c:c + 1] for c in range(3)]
    x2c = [x2t[c:c + 1, :] for c in range(3)]
    cross = jax.lax.dot_general(
        x1, x2t, (((1,), (0,)), ((), ())),
        preferred_element_type=jnp.float32)
    x1sq = x1c[0] * x1c[0] + x1c[1] * x1c[1] + x1c[2] * x1c[2]
    x2sq = x2c[0] * x2c[0] + x2c[1] * x2c[1] + x2c[2] * x2c[2]
    d = (-2.0) * cross + x1sq + x2sq

    iota = lax.broadcasted_iota(jnp.int32, (NT1, S), 1)
    wmat = jnp.zeros((NT1, S), dtype=jnp.float32)
    recips = []
    dcur = d
    for _ in range(3):
        m = jnp.min(dcur, axis=1, keepdims=True)
        idx = jnp.min(jnp.where(dcur == m, iota, S), axis=1, keepdims=True)
        hit = iota == idx
        recips.append((m, hit))
        dcur = jnp.where(hit, jnp.float32(jnp.inf), dcur)
    norm = sum(1.0 / (m + 1e-8) for m, _ in recips)
    for m, hit in recips:
        w = (1.0 / (m + 1e-8)) / norm
        wmat = wmat + jnp.where(hit, w, 0.0)

    interp = jax.lax.dot_general(
        wmat, p2_ref[0], (((1,), (0,)), ((), ())),
        preferred_element_type=jnp.float32,
        precision=jax.lax.Precision.HIGHEST)

    h0 = jax.lax.dot_general(
        p1_ref[0], w0a_ref[...], (((1,), (0,)), ((), ())),
        preferred_element_type=jnp.float32,
        precision=jax.lax.Precision.HIGHEST)
    h0 = h0 + jax.lax.dot_general(
        interp, w0b_ref[...], (((1,), (0,)), ((), ())),
        preferred_element_type=jnp.float32,
        precision=jax.lax.Precision.HIGHEST)
    h0 = h0 + b0_ref[...]
    h0_ref[0] = h0

    first = (pl.program_id(0) == 0) & (pl.program_id(1) == 0)

    @pl.when(first)
    def _():
        s0_ref[...] = jnp.zeros_like(s0_ref)
        q0_ref[...] = jnp.zeros_like(q0_ref)

    s0_ref[...] += jnp.sum(h0, axis=0, keepdims=True)
    q0_ref[...] += jnp.sum(h0 * h0, axis=0, keepdims=True)


def _k2_body(h0_ref, a0_ref, c0_ref, w1t_ref, b1_ref,
             h2_ref, s1_ref, q1_ref):
    h1 = jnp.maximum(h0_ref[...] * a0_ref[...] + c0_ref[...], 0.0)
    h2 = jax.lax.dot_general(
        h1, w1t_ref[...], (((1,), (0,)), ((), ())),
        preferred_element_type=jnp.float32,
        precision=jax.lax.Precision.HIGHEST) + b1_ref[...]
    h2_ref[...] = h2

    @pl.when(pl.program_id(0) == 0)
    def _():
        s1_ref[...] = jnp.zeros_like(s1_ref)
        q1_ref[...] = jnp.zeros_like(q1_ref)

    s1_ref[...] += jnp.sum(h2, axis=0, keepdims=True)
    q1_ref[...] += jnp.sum(h2 * h2, axis=0, keepdims=True)


def _k3_body(h2_ref, a1_ref, c1_ref, out_ref):
    out_ref[...] = jnp.maximum(h2_ref[...] * a1_ref[...] + c1_ref[...], 0.0)


def kernel(xyz1, xyz2, points1, points2, W0, b0, gamma0, beta0,
           W1, b1, gamma1, beta1):
    B, N, _ = xyz1.shape
    S = xyz2.shape[1]
    D1 = points1.shape[2]
    C0 = W0.shape[0]
    C1 = W1.shape[0]
    nt = N // NT1
    count = B * N

    w0a = W0[:, :D1].T
    w0b = W0[:, D1:].T
    w1t = W1.T
    xyz2t = jnp.transpose(xyz2, (0, 2, 1))

    h0, s0, q0 = pl.pallas_call(
        _k1_body,
        grid=(B, nt),
        in_specs=[
            pl.BlockSpec((1, NT1, 3), lambda b, i: (b, i, 0)),
            pl.BlockSpec((1, 3, S), lambda b, i: (b, 0, 0)),
            pl.BlockSpec((1, NT1, D1), lambda b, i: (b, i, 0)),
            pl.BlockSpec((1, S, points2.shape[2]), lambda b, i: (b, 0, 0)),
            pl.BlockSpec((D1, C0), lambda b, i: (0, 0)),
            pl.BlockSpec((points2.shape[2], C0), lambda b, i: (0, 0)),
            pl.BlockSpec((1, C0), lambda b, i: (0, 0)),
        ],
        out_specs=[
            pl.BlockSpec((1, NT1, C0), lambda b, i: (b, i, 0)),
            pl.BlockSpec((1, C0), lambda b, i: (0, 0)),
            pl.BlockSpec((1, C0), lambda b, i: (0, 0)),
        ],
        out_shape=[
            jax.ShapeDtypeStruct((B, N, C0), jnp.float32),
            jax.ShapeDtypeStruct((1, C0), jnp.float32),
            jax.ShapeDtypeStruct((1, C0), jnp.float32),
        ],
    )(xyz1, xyz2t, points1, points2, w0a, w0b, b0.reshape(1, C0))

    mean0 = s0 / count
    var0 = q0 / count - mean0 * mean0
    a0 = gamma0.reshape(1, C0) / jnp.sqrt(var0 + 1e-5)
    c0 = beta0.reshape(1, C0) - mean0 * a0

    h0f = h0.reshape(B * N, C0)
    h2, s1, q1 = pl.pallas_call(
        _k2_body,
        grid=(count // NT2,),
        in_specs=[
            pl.BlockSpec((NT2, C0), lambda i: (i, 0)),
            pl.BlockSpec((1, C0), lambda i: (0, 0)),
            pl.BlockSpec((1, C0), lambda i: (0, 0)),
            pl.BlockSpec((C0, C1), lambda i: (0, 0)),
            pl.BlockSpec((1, C1), lambda i: (0, 0)),
        ],
        out_specs=[
            pl.BlockSpec((NT2, C1), lambda i: (i, 0)),
            pl.BlockSpec((1, C1), lambda i: (0, 0)),
            pl.BlockSpec((1, C1), lambda i: (0, 0)),
        ],
        out_shape=[
            jax.ShapeDtypeStruct((count, C1), jnp.float32),
            jax.ShapeDtypeStruct((1, C1), jnp.float32),
            jax.ShapeDtypeStruct((1, C1), jnp.float32),
        ],
    )(h0f, a0, c0, w1t, b1.reshape(1, C1))

    mean1 = s1 / count
    var1 = q1 / count - mean1 * mean1
    a1 = gamma1.reshape(1, C1) / jnp.sqrt(var1 + 1e-5)
    c1 = beta1.reshape(1, C1) - mean1 * a1

    out = pl.pallas_call(
        _k3_body,
        grid=(count // NT2,),
        in_specs=[
            pl.BlockSpec((NT2, C1), lambda i: (i, 0)),
            pl.BlockSpec((1, C1), lambda i: (0, 0)),
            pl.BlockSpec((1, C1), lambda i: (0, 0)),
        ],
        out_specs=pl.BlockSpec((NT2, C1), lambda i: (i, 0)),
        out_shape=jax.ShapeDtypeStruct((count, C1), jnp.float32),
    )(h2, a1, c1)

    return out.reshape(B, N, C1)


# speedup vs baseline: 25.4152x; 1.4451x over previous
import functools

import jax
import jax.numpy as jnp
from jax import lax
from jax.experimental import pallas as pl
from jax.experimental.pallas import tpu as pltpu
from jax.experimental.pallas import tpu_sc as plsc


NT1 = 256
NT2 = 512
NW = 32
PC = 128


def _k1_body(x1_ref, x2t_ref, w_ref, gi_ref):
    S = x2t_ref.shape[2]
    x1 = x1_ref[0]
    x2t = x2t_ref[0]
    cross = jax.lax.dot_general(
        x1, x2t, (((1,), (0,)), ((), ())),
        preferred_element_type=jnp.float32)
    x1c = [x1[:, c:c + 1] for c in range(3)]
    x2c = [x2t[c:c + 1, :] for c in range(3)]
    x1sq = x1c[0] * x1c[0] + x1c[1] * x1c[1] + x1c[2] * x1c[2]
    x2sq = x2c[0] * x2c[0] + x2c[1] * x2c[1] + x2c[2] * x2c[2]
    d = (-2.0) * cross + x1sq + x2sq

    iota = lax.broadcasted_iota(jnp.int32, (NT1, S), 1)
    ms, idxs = [], []
    dcur = d
    for _ in range(3):
        m = jnp.min(dcur, axis=1, keepdims=True)
        hit = dcur == m
        idx = jnp.min(jnp.where(hit, iota, S), axis=1, keepdims=True)
        ms.append(m)
        idxs.append(idx)
        dcur = jnp.where(iota == idx, jnp.float32(jnp.inf), dcur)
    r = [1.0 / (m + 1e-8) for m in ms]
    norm = r[0] + r[1] + r[2]
    w_ref[0] = jnp.concatenate([rk / norm for rk in r], axis=1)
    base = pl.program_id(0) * S
    gi_ref[0] = jnp.concatenate(idxs, axis=1) + base


def _sc_interp_body(p2_hbm, gidx_hbm, w_hbm, out_hbm,
                    idx_v, w_v, rows_v, out_v, sem):
    wid = lax.axis_index("s") * 2 + lax.axis_index("c")
    npts = out_hbm.shape[0]
    pw = npts // NW

    def chunk_body(ci, _):
        base = wid * pw + ci * PC
        ib = base * 3
        pltpu.sync_copy(gidx_hbm.at[pl.ds(ib, 3 * PC)], idx_v)
        pltpu.sync_copy(w_hbm.at[pl.ds(ib, 3 * PC)], w_v)
        cps = []
        for k in range(3):
            cps.append(pltpu.async_copy(
                p2_hbm.at[idx_v.at[pl.ds(128 * k, 128)]],
                rows_v.at[pl.ds(128 * k, 128)], sem))
        for cp in cps:
            cp.wait()

        def group_body(g, _):
            wvecs = [w_v[pl.ds(48 * g + 16 * j, 16)] for j in range(3)]
            p0 = 16 * g
            for i in range(16):
                w0 = wvecs[(3 * i) // 16][(3 * i) % 16]
                w1 = wvecs[(3 * i + 1) // 16][(3 * i + 1) % 16]
                w2 = wvecs[(3 * i + 2) // 16][(3 * i + 2) % 16]
                p = p0 + i
                for c in range(4):
                    sl = pl.ds(16 * c, 16)
                    out_v[p, sl] = (w0 * rows_v[3 * p, sl]
                                    + w1 * rows_v[3 * p + 1, sl]
                                    ) + w2 * rows_v[3 * p + 2, sl]
            return 0

        lax.fori_loop(0, PC // 16, group_body, 0)
        pltpu.sync_copy(out_v, out_hbm.at[pl.ds(base, PC)])
        return 0

    lax.fori_loop(0, pw // PC, chunk_body, 0)


def _k3_body(p1_ref, it_ref, w0a_ref, w0b_ref, b0_ref, h0_ref, s0_ref, q0_ref):
    h0 = jax.lax.dot_general(
        p1_ref[...], w0a_ref[...], (((1,), (0,)), ((), ())),
        preferred_element_type=jnp.float32)
    h0 = h0 + jax.lax.dot_general(
        it_ref[...], w0b_ref[...], (((1,), (0,)), ((), ())),
        preferred_element_type=jnp.float32)
    h0 = h0 + b0_ref[...]
    h0_ref[...] = h0

    @pl.when(pl.program_id(0) == 0)
    def _():
        s0_ref[...] = jnp.zeros_like(s0_ref)
        q0_ref[...] = jnp.zeros_like(q0_ref)

    s0_ref[...] += jnp.sum(h0, axis=0, keepdims=True)
    q0_ref[...] += jnp.sum(h0 * h0, axis=0, keepdims=True)


def _k4_body(h0_ref, a0_ref, c0_ref, w1t_ref, b1_ref,
             h2_ref, s1_ref, q1_ref):
    h1 = jnp.maximum(h0_ref[...] * a0_ref[...] + c0_ref[...], 0.0)
    h2 = jax.lax.dot_general(
        h1, w1t_ref[...], (((1,), (0,)), ((), ())),
        preferred_element_type=jnp.float32) + b1_ref[...]
    h2_ref[...] = h2

    @pl.when(pl.program_id(0) == 0)
    def _():
        s1_ref[...] = jnp.zeros_like(s1_ref)
        q1_ref[...] = jnp.zeros_like(q1_ref)

    s1_ref[...] += jnp.sum(h2, axis=0, keepdims=True)
    q1_ref[...] += jnp.sum(h2 * h2, axis=0, keepdims=True)


def _k5_body(h2_ref, a1_ref, c1_ref, out_ref):
    out_ref[...] = jnp.maximum(h2_ref[...] * a1_ref[...] + c1_ref[...], 0.0)


def kernel(xyz1, xyz2, points1, points2, W0, b0, gamma0, beta0,
           W1, b1, gamma1, beta1):
    B, N, _ = xyz1.shape
    S = xyz2.shape[1]
    D1 = points1.shape[2]
    D2 = points2.shape[2]
    C0 = W0.shape[0]
    C1 = W1.shape[0]
    nt = N // NT1
    count = B * N

    w0a = W0[:, :D1].T
    w0b = W0[:, D1:].T
    w1t = W1.T
    xyz2t = jnp.transpose(xyz2, (0, 2, 1))

    w3, gi3 = pl.pallas_call(
        _k1_body,
        grid=(B, nt),
        in_specs=[
            pl.BlockSpec((1, NT1, 3), lambda b, i: (b, i, 0)),
            pl.BlockSpec((1, 3, S), lambda b, i: (b, 0, 0)),
        ],
        out_specs=[
            pl.BlockSpec((1, NT1, 3), lambda b, i: (b, i, 0)),
            pl.BlockSpec((1, NT1, 3), lambda b, i: (b, i, 0)),
        ],
        out_shape=[
            jax.ShapeDtypeStruct((B, N, 3), jnp.float32),
            jax.ShapeDtypeStruct((B, N, 3), jnp.int32),
        ],
    )(xyz1, xyz2t)

    p2pad = jnp.concatenate(
        [points2.reshape(B * S, D2),
         jnp.zeros((B * S, 128 - D2), jnp.float32)], axis=1)
    interp = pl.kernel(
        _sc_interp_body,
        mesh=plsc.VectorSubcoreMesh(core_axis_name="c", subcore_axis_name="s"),
        out_type=jax.ShapeDtypeStruct((count, D2), jnp.float32),
        scratch_types=[
            pltpu.VMEM((3 * PC,), jnp.int32),
            pltpu.VMEM((3 * PC,), jnp.float32),
            pltpu.VMEM((3 * PC, 128), jnp.float32),
            pltpu.VMEM((PC, D2), jnp.float32),
            pltpu.SemaphoreType.DMA,
        ],
    )(p2pad, gi3.reshape(count * 3), w3.reshape(count * 3))

    p1flat = points1.reshape(count, D1)
    h0f, s0, q0 = pl.pallas_call(
        _k3_body,
        grid=(count // NT2,),
        in_specs=[
            pl.BlockSpec((NT2, D1), lambda i: (i, 0)),
            pl.BlockSpec((NT2, D2), lambda i: (i, 0)),
            pl.BlockSpec((D1, C0), lambda i: (0, 0)),
            pl.BlockSpec((D2, C0), lambda i: (0, 0)),
            pl.BlockSpec((1, C0), lambda i: (0, 0)),
        ],
        out_specs=[
            pl.BlockSpec((NT2, C0), lambda i: (i, 0)),
            pl.BlockSpec((1, C0), lambda i: (0, 0)),
            pl.BlockSpec((1, C0), lambda i: (0, 0)),
        ],
        out_shape=[
            jax.ShapeDtypeStruct((count, C0), jnp.float32),
            jax.ShapeDtypeStruct((1, C0), jnp.float32),
            jax.ShapeDtypeStruct((1, C0), jnp.float32),
        ],
    )(p1flat, interp, w0a, w0b, b0.reshape(1, C0))

    mean0 = s0 / count
    var0 = q0 / count - mean0 * mean0
    a0 = gamma0.reshape(1, C0) / jnp.sqrt(var0 + 1e-5)
    c0 = beta0.reshape(1, C0) - mean0 * a0

    h2, s1, q1 = pl.pallas_call(
        _k4_body,
        grid=(count // NT2,),
        in_specs=[
            pl.BlockSpec((NT2, C0), lambda i: (i, 0)),
            pl.BlockSpec((1, C0), lambda i: (0, 0)),
            pl.BlockSpec((1, C0), lambda i: (0, 0)),
            pl.BlockSpec((C0, C1), lambda i: (0, 0)),
            pl.BlockSpec((1, C1), lambda i: (0, 0)),
        ],
        out_specs=[
            pl.BlockSpec((NT2, C1), lambda i: (i, 0)),
            pl.BlockSpec((1, C1), lambda i: (0, 0)),
            pl.BlockSpec((1, C1), lambda i: (0, 0)),
        ],
        out_shape=[
            jax.ShapeDtypeStruct((count, C1), jnp.float32),
            jax.ShapeDtypeStruct((1, C1), jnp.float32),
            jax.ShapeDtypeStruct((1, C1), jnp.float32),
        ],
    )(h0f, a0, c0, w1t, b1.reshape(1, C1))

    mean1 = s1 / count
    var1 = q1 / count - mean1 * mean1
    a1 = gamma1.reshape(1, C1) / jnp.sqrt(var1 + 1e-5)
    c1 = beta1.reshape(1, C1) - mean1 * a1

    out = pl.pallas_call(
        _k5_body,
        grid=(count // NT2,),
        in_specs=[
            pl.BlockSpec((NT2, C1), lambda i: (i, 0)),
            pl.BlockSpec((1, C1), lambda i: (0, 0)),
            pl.BlockSpec((1, C1), lambda i: (0, 0)),
        ],
        out_specs=pl.BlockSpec((NT2, C1), lambda i: (i, 0)),
        out_shape=jax.ShapeDtypeStruct((count, C1), jnp.float32),
    )(h2, a1, c1)

    return out.reshape(B, N, C1)


# speedup vs baseline: 27.0127x; 1.0629x over previous
import functools

import jax
import jax.numpy as jnp
from jax import lax
from jax.experimental import pallas as pl
from jax.experimental.pallas import tpu as pltpu
from jax.experimental.pallas import tpu_sc as plsc


NT1 = 512
NT2 = 512
NW = 32
PC = 128


def _k1_body(x1_ref, x2t_ref, w_ref, gi_ref):
    S = x2t_ref.shape[2]
    x1 = x1_ref[0]
    x2t = x2t_ref[0]
    cross = jax.lax.dot_general(
        x1, x2t, (((1,), (0,)), ((), ())),
        preferred_element_type=jnp.float32)
    x1c = [x1[:, c:c + 1] for c in range(3)]
    x2c = [x2t[c:c + 1, :] for c in range(3)]
    x1sq = x1c[0] * x1c[0] + x1c[1] * x1c[1] + x1c[2] * x1c[2]
    x2sq = x2c[0] * x2c[0] + x2c[1] * x2c[1] + x2c[2] * x2c[2]
    d = (-2.0) * cross + x1sq + x2sq

    iota = lax.broadcasted_iota(jnp.int32, (NT1, S), 1)
    ms, idxs = [], []
    dcur = d
    for _ in range(3):
        m = jnp.min(dcur, axis=1, keepdims=True)
        hit = dcur == m
        idx = jnp.min(jnp.where(hit, iota, S), axis=1, keepdims=True)
        ms.append(m)
        idxs.append(idx)
        dcur = jnp.where(iota == idx, jnp.float32(jnp.inf), dcur)
    r = [1.0 / (m + 1e-8) for m in ms]
    norm = r[0] + r[1] + r[2]
    w_ref[0] = jnp.concatenate([rk / norm for rk in r], axis=1)
    base = pl.program_id(0) * S
    gi_ref[0] = jnp.concatenate(idxs, axis=1) + base


def _sc_interp_body(p2_hbm, gidx_hbm, w_hbm, out_hbm,
                    idx_v, w_v, rows_v, out_v, sem):
    wid = lax.axis_index("s") * 2 + lax.axis_index("c")
    npts = out_hbm.shape[0]
    pw = npts // NW

    def chunk_body(ci, _):
        base = wid * pw + ci * PC
        ib = base * 3
        pltpu.sync_copy(gidx_hbm.at[pl.ds(ib, 3 * PC)], idx_v)
        pltpu.sync_copy(w_hbm.at[pl.ds(ib, 3 * PC)], w_v)
        cps = []
        for k in range(3):
            cps.append(pltpu.async_copy(
                p2_hbm.at[idx_v.at[pl.ds(128 * k, 128)]],
                rows_v.at[pl.ds(128 * k, 128)], sem))
        for cp in cps:
            cp.wait()

        def group_body(g, _):
            wvecs = [w_v[pl.ds(48 * g + 16 * j, 16)] for j in range(3)]
            p0 = 16 * g
            for i in range(16):
                w0 = wvecs[(3 * i) // 16][(3 * i) % 16]
                w1 = wvecs[(3 * i + 1) // 16][(3 * i + 1) % 16]
                w2 = wvecs[(3 * i + 2) // 16][(3 * i + 2) % 16]
                p = p0 + i
                for c in range(4):
                    sl = pl.ds(16 * c, 16)
                    out_v[p, sl] = (w0 * rows_v[3 * p, sl]
                                    + w1 * rows_v[3 * p + 1, sl]
                                    ) + w2 * rows_v[3 * p + 2, sl]
            return 0

        lax.fori_loop(0, PC // 16, group_body, 0)
        pltpu.sync_copy(out_v, out_hbm.at[pl.ds(base, PC)])
        return 0

    lax.fori_loop(0, pw // PC, chunk_body, 0)


def _k3_body(p1_ref, it_ref, w0a_ref, w0b_ref, b0_ref, h0_ref, s0_ref, q0_ref):
    h0 = jax.lax.dot_general(
        p1_ref[...], w0a_ref[...], (((1,), (0,)), ((), ())),
        preferred_element_type=jnp.float32)
    h0 = h0 + jax.lax.dot_general(
        it_ref[...], w0b_ref[...], (((1,), (0,)), ((), ())),
        preferred_element_type=jnp.float32)
    h0 = h0 + b0_ref[...]
    h0_ref[...] = h0

    @pl.when(pl.program_id(0) == 0)
    def _():
        s0_ref[...] = jnp.zeros_like(s0_ref)
        q0_ref[...] = jnp.zeros_like(q0_ref)

    s0_ref[...] += jnp.sum(h0, axis=0, keepdims=True)
    q0_ref[...] += jnp.sum(h0 * h0, axis=0, keepdims=True)


def _k4_body(h0_ref, a0_ref, c0_ref, w1t_ref, b1_ref,
             h2_ref, s1_ref, q1_ref):
    h1 = jnp.maximum(h0_ref[...] * a0_ref[...] + c0_ref[...], 0.0)
    h2 = jax.lax.dot_general(
        h1, w1t_ref[...], (((1,), (0,)), ((), ())),
        preferred_element_type=jnp.float32) + b1_ref[...]
    h2_ref[...] = h2

    @pl.when(pl.program_id(0) == 0)
    def _():
        s1_ref[...] = jnp.zeros_like(s1_ref)
        q1_ref[...] = jnp.zeros_like(q1_ref)

    s1_ref[...] += jnp.sum(h2, axis=0, keepdims=True)
    q1_ref[...] += jnp.sum(h2 * h2, axis=0, keepdims=True)


def _k5_body(h2_ref, a1_ref, c1_ref, out_ref):
    out_ref[...] = jnp.maximum(h2_ref[...] * a1_ref[...] + c1_ref[...], 0.0)


def kernel(xyz1, xyz2, points1, points2, W0, b0, gamma0, beta0,
           W1, b1, gamma1, beta1):
    B, N, _ = xyz1.shape
    S = xyz2.shape[1]
    D1 = points1.shape[2]
    D2 = points2.shape[2]
    C0 = W0.shape[0]
    C1 = W1.shape[0]
    nt = N // NT1
    count = B * N

    w0a = W0[:, :D1].T
    w0b = W0[:, D1:].T
    w1t = W1.T
    xyz2t = jnp.transpose(xyz2, (0, 2, 1))

    w3, gi3 = pl.pallas_call(
        _k1_body,
        grid=(B, nt),
        in_specs=[
            pl.BlockSpec((1, NT1, 3), lambda b, i: (b, i, 0)),
            pl.BlockSpec((1, 3, S), lambda b, i: (b, 0, 0)),
        ],
        out_specs=[
            pl.BlockSpec((1, NT1, 3), lambda b, i: (b, i, 0)),
            pl.BlockSpec((1, NT1, 3), lambda b, i: (b, i, 0)),
        ],
        out_shape=[
            jax.ShapeDtypeStruct((B, N, 3), jnp.float32),
            jax.ShapeDtypeStruct((B, N, 3), jnp.int32),
        ],
    )(xyz1, xyz2t)

    p2pad = jnp.concatenate(
        [points2.reshape(B * S, D2),
         jnp.zeros((B * S, 128 - D2), jnp.float32)], axis=1)
    interp = pl.kernel(
        _sc_interp_body,
        mesh=plsc.VectorSubcoreMesh(core_axis_name="c", subcore_axis_name="s"),
        out_type=jax.ShapeDtypeStruct((count, D2), jnp.float32),
        scratch_types=[
            pltpu.VMEM((3 * PC,), jnp.int32),
            pltpu.VMEM((3 * PC,), jnp.float32),
            pltpu.VMEM((3 * PC, 128), jnp.float32),
            pltpu.VMEM((PC, D2), jnp.float32),
            pltpu.SemaphoreType.DMA,
        ],
    )(p2pad, gi3.reshape(count * 3), w3.reshape(count * 3))

    p1flat = points1.reshape(count, D1)
    h0f, s0, q0 = pl.pallas_call(
        _k3_body,
        grid=(count // NT2,),
        in_specs=[
            pl.BlockSpec((NT2, D1), lambda i: (i, 0)),
            pl.BlockSpec((NT2, D2), lambda i: (i, 0)),
            pl.BlockSpec((D1, C0), lambda i: (0, 0)),
            pl.BlockSpec((D2, C0), lambda i: (0, 0)),
            pl.BlockSpec((1, C0), lambda i: (0, 0)),
        ],
        out_specs=[
            pl.BlockSpec((NT2, C0), lambda i: (i, 0)),
            pl.BlockSpec((1, C0), lambda i: (0, 0)),
            pl.BlockSpec((1, C0), lambda i: (0, 0)),
        ],
        out_shape=[
            jax.ShapeDtypeStruct((count, C0), jnp.float32),
            jax.ShapeDtypeStruct((1, C0), jnp.float32),
            jax.ShapeDtypeStruct((1, C0), jnp.float32),
        ],
    )(p1flat, interp, w0a, w0b, b0.reshape(1, C0))

    mean0 = s0 / count
    var0 = q0 / count - mean0 * mean0
    a0 = gamma0.reshape(1, C0) / jnp.sqrt(var0 + 1e-5)
    c0 = beta0.reshape(1, C0) - mean0 * a0

    h2, s1, q1 = pl.pallas_call(
        _k4_body,
        grid=(count // NT2,),
        in_specs=[
            pl.BlockSpec((NT2, C0), lambda i: (i, 0)),
            pl.BlockSpec((1, C0), lambda i: (0, 0)),
            pl.BlockSpec((1, C0), lambda i: (0, 0)),
            pl.BlockSpec((C0, C1), lambda i: (0, 0)),
            pl.BlockSpec((1, C1), lambda i: (0, 0)),
        ],
        out_specs=[
            pl.BlockSpec((NT2, C1), lambda i: (i, 0)),
            pl.BlockSpec((1, C1), lambda i: (0, 0)),
            pl.BlockSpec((1, C1), lambda i: (0, 0)),
        ],
        out_shape=[
            jax.ShapeDtypeStruct((count, C1), jnp.float32),
            jax.ShapeDtypeStruct((1, C1), jnp.float32),
            jax.ShapeDtypeStruct((1, C1), jnp.float32),
        ],
    )(h0f, a0, c0, w1t, b1.reshape(1, C1))

    mean1 = s1 / count
    var1 = q1 / count - mean1 * mean1
    a1 = gamma1.reshape(1, C1) / jnp.sqrt(var1 + 1e-5)
    c1 = beta1.reshape(1, C1) - mean1 * a1

    out = pl.pallas_call(
        _k5_body,
        grid=(count // NT2,),
        in_specs=[
            pl.BlockSpec((NT2, C1), lambda i: (i, 0)),
            pl.BlockSpec((1, C1), lambda i: (0, 0)),
            pl.BlockSpec((1, C1), lambda i: (0, 0)),
        ],
        out_specs=pl.BlockSpec((NT2, C1), lambda i: (i, 0)),
        out_shape=jax.ShapeDtypeStruct((count, C1), jnp.float32),
    )(h2, a1, c1)

    return out.reshape(B, N, C1)


# speedup vs baseline: 27.9687x; 1.0354x over previous
import functools

import jax
import jax.numpy as jnp
from jax import lax
from jax.experimental import pallas as pl
from jax.experimental.pallas import tpu as pltpu
from jax.experimental.pallas import tpu_sc as plsc


NT1 = 512
NT2 = 512
NW = 32
PC = 128


def _k1_body(gbase, x1_ref, x2t_ref, w_ref, gi_ref):
    S = x2t_ref.shape[1]
    x1 = x1_ref[...]
    x2t = x2t_ref[...]
    cross = jax.lax.dot_general(
        x1, x2t, (((1,), (0,)), ((), ())),
        preferred_element_type=jnp.float32)
    x1c = [x1[:, c:c + 1] for c in range(3)]
    x2c = [x2t[c:c + 1, :] for c in range(3)]
    x1sq = x1c[0] * x1c[0] + x1c[1] * x1c[1] + x1c[2] * x1c[2]
    x2sq = x2c[0] * x2c[0] + x2c[1] * x2c[1] + x2c[2] * x2c[2]
    d = (-2.0) * cross + x1sq + x2sq

    iota = lax.broadcasted_iota(jnp.int32, (NT1, S), 1)
    ms, idxs = [], []
    dcur = d
    for _ in range(3):
        m = jnp.min(dcur, axis=1, keepdims=True)
        hit = dcur == m
        idx = jnp.min(jnp.where(hit, iota, S), axis=1, keepdims=True)
        ms.append(m)
        idxs.append(idx)
        dcur = jnp.where(iota == idx, jnp.float32(jnp.inf), dcur)
    r = [1.0 / (m + 1e-8) for m in ms]
    norm = r[0] + r[1] + r[2]
    w_ref[...] = jnp.concatenate([rk / norm for rk in r], axis=1)
    gi_ref[...] = jnp.concatenate(idxs, axis=1) + gbase


def _sc_interp_body(p2_hbm, gidx_hbm, w_hbm, out_hbm,
                    idx_v, w_v, rows_v, out_v, sem):
    wid = lax.axis_index("s") * 2 + lax.axis_index("c")
    npts = out_hbm.shape[0]
    pw = npts // NW

    def chunk_body(ci, _):
        base = wid * pw + ci * PC
        ib = base * 3
        pltpu.sync_copy(gidx_hbm.at[pl.ds(ib, 3 * PC)], idx_v)
        pltpu.sync_copy(w_hbm.at[pl.ds(ib, 3 * PC)], w_v)
        cps = []
        for k in range(3):
            cps.append(pltpu.async_copy(
                p2_hbm.at[idx_v.at[pl.ds(128 * k, 128)]],
                rows_v.at[pl.ds(128 * k, 128)], sem))
        for cp in cps:
            cp.wait()

        def group_body(g, _):
            wvecs = [w_v[pl.ds(48 * g + 16 * j, 16)] for j in range(3)]
            p0 = 16 * g
            for i in range(16):
                w0 = wvecs[(3 * i) // 16][(3 * i) % 16]
                w1 = wvecs[(3 * i + 1) // 16][(3 * i + 1) % 16]
                w2 = wvecs[(3 * i + 2) // 16][(3 * i + 2) % 16]
                p = p0 + i
                for c in range(4):
                    sl = pl.ds(16 * c, 16)
                    out_v[p, sl] = (w0 * rows_v[3 * p, sl]
                                    + w1 * rows_v[3 * p + 1, sl]
                                    ) + w2 * rows_v[3 * p + 2, sl]
            return 0

        lax.fori_loop(0, PC // 16, group_body, 0)
        pltpu.sync_copy(out_v, out_hbm.at[pl.ds(base, PC)])
        return 0

    lax.fori_loop(0, pw // PC, chunk_body, 0)


def _k3_body(p1_ref, it_ref, w0a_ref, w0b_ref, b0_ref, h0_ref, s0_ref, q0_ref):
    h0 = jax.lax.dot_general(
        p1_ref[...], w0a_ref[...], (((1,), (0,)), ((), ())),
        preferred_element_type=jnp.float32)
    h0 = h0 + jax.lax.dot_general(
        it_ref[...], w0b_ref[...], (((1,), (0,)), ((), ())),
        preferred_element_type=jnp.float32)
    h0 = h0 + b0_ref[...]
    h0_ref[...] = h0

    @pl.when(pl.program_id(0) == 0)
    def _():
        s0_ref[...] = jnp.zeros_like(s0_ref)
        q0_ref[...] = jnp.zeros_like(q0_ref)

    s0_ref[...] += jnp.sum(h0, axis=0, keepdims=True)
    q0_ref[...] += jnp.sum(h0 * h0, axis=0, keepdims=True)


def _k4_body(h0_ref, a0_ref, c0_ref, w1t_ref, b1_ref,
             h2_ref, s1_ref, q1_ref):
    h1 = jnp.maximum(h0_ref[...] * a0_ref[...] + c0_ref[...], 0.0)
    h2 = jax.lax.dot_general(
        h1, w1t_ref[...], (((1,), (0,)), ((), ())),
        preferred_element_type=jnp.float32) + b1_ref[...]
    h2_ref[...] = h2

    @pl.when(pl.program_id(0) == 0)
    def _():
        s1_ref[...] = jnp.zeros_like(s1_ref)
        q1_ref[...] = jnp.zeros_like(q1_ref)

    s1_ref[...] += jnp.sum(h2, axis=0, keepdims=True)
    q1_ref[...] += jnp.sum(h2 * h2, axis=0, keepdims=True)


def _k5_body(h2_ref, a1_ref, c1_ref, out_ref):
    out_ref[...] = jnp.maximum(h2_ref[...] * a1_ref[...] + c1_ref[...], 0.0)


def kernel(xyz1, xyz2, points1, points2, W0, b0, gamma0, beta0,
           W1, b1, gamma1, beta1):
    B, N, _ = xyz1.shape
    S = xyz2.shape[1]
    D1 = points1.shape[2]
    D2 = points2.shape[2]
    C0 = W0.shape[0]
    C1 = W1.shape[0]
    nt = N // NT1
    count = B * N

    w0a = W0[:, :D1].T
    w0b = W0[:, D1:].T
    w1t = W1.T
    xyz2t = jnp.transpose(xyz2, (0, 2, 1))

    p2pad = jnp.concatenate(
        [points2.reshape(B * S, D2),
         jnp.zeros((B * S, 128 - D2), jnp.float32)], axis=1)

    sc_mesh = plsc.VectorSubcoreMesh(core_axis_name="c", subcore_axis_name="s")
    interp_parts = []
    for b in range(B):
        w3, gi3 = pl.pallas_call(
            functools.partial(_k1_body, b * S),
            grid=(nt,),
            in_specs=[
                pl.BlockSpec((NT1, 3), lambda i: (i, 0)),
                pl.BlockSpec((3, S), lambda i: (0, 0)),
            ],
            out_specs=[
                pl.BlockSpec((NT1, 3), lambda i: (i, 0)),
                pl.BlockSpec((NT1, 3), lambda i: (i, 0)),
            ],
            out_shape=[
                jax.ShapeDtypeStruct((N, 3), jnp.float32),
                jax.ShapeDtypeStruct((N, 3), jnp.int32),
            ],
        )(xyz1[b], xyz2t[b])
        interp_parts.append(pl.kernel(
            _sc_interp_body,
            mesh=sc_mesh,
            out_type=jax.ShapeDtypeStruct((N, D2), jnp.float32),
            scratch_types=[
                pltpu.VMEM((3 * PC,), jnp.int32),
                pltpu.VMEM((3 * PC,), jnp.float32),
                pltpu.VMEM((3 * PC, 128), jnp.float32),
                pltpu.VMEM((PC, D2), jnp.float32),
                pltpu.SemaphoreType.DMA,
            ],
        )(p2pad, gi3.reshape(N * 3), w3.reshape(N * 3)))
    interp = jnp.concatenate(interp_parts, axis=0)

    p1flat = points1.reshape(count, D1)
    h0f, s0, q0 = pl.pallas_call(
        _k3_body,
        grid=(count // NT2,),
        in_specs=[
            pl.BlockSpec((NT2, D1), lambda i: (i, 0)),
            pl.BlockSpec((NT2, D2), lambda i: (i, 0)),
            pl.BlockSpec((D1, C0), lambda i: (0, 0)),
            pl.BlockSpec((D2, C0), lambda i: (0, 0)),
            pl.BlockSpec((1, C0), lambda i: (0, 0)),
        ],
        out_specs=[
            pl.BlockSpec((NT2, C0), lambda i: (i, 0)),
            pl.BlockSpec((1, C0), lambda i: (0, 0)),
            pl.BlockSpec((1, C0), lambda i: (0, 0)),
        ],
        out_shape=[
            jax.ShapeDtypeStruct((count, C0), jnp.float32),
            jax.ShapeDtypeStruct((1, C0), jnp.float32),
            jax.ShapeDtypeStruct((1, C0), jnp.float32),
        ],
    )(p1flat, interp, w0a, w0b, b0.reshape(1, C0))

    mean0 = s0 / count
    var0 = q0 / count - mean0 * mean0
    a0 = gamma0.reshape(1, C0) / jnp.sqrt(var0 + 1e-5)
    c0 = beta0.reshape(1, C0) - mean0 * a0

    h2, s1, q1 = pl.pallas_call(
        _k4_body,
        grid=(count // NT2,),
        in_specs=[
            pl.BlockSpec((NT2, C0), lambda i: (i, 0)),
            pl.BlockSpec((1, C0), lambda i: (0, 0)),
            pl.BlockSpec((1, C0), lambda i: (0, 0)),
            pl.BlockSpec((C0, C1), lambda i: (0, 0)),
            pl.BlockSpec((1, C1), lambda i: (0, 0)),
        ],
        out_specs=[
            pl.BlockSpec((NT2, C1), lambda i: (i, 0)),
            pl.BlockSpec((1, C1), lambda i: (0, 0)),
            pl.BlockSpec((1, C1), lambda i: (0, 0)),
        ],
        out_shape=[
            jax.ShapeDtypeStruct((count, C1), jnp.float32),
            jax.ShapeDtypeStruct((1, C1), jnp.float32),
            jax.ShapeDtypeStruct((1, C1), jnp.float32),
        ],
    )(h0f, a0, c0, w1t, b1.reshape(1, C1))

    mean1 = s1 / count
    var1 = q1 / count - mean1 * mean1
    a1 = gamma1.reshape(1, C1) / jnp.sqrt(var1 + 1e-5)
    c1 = beta1.reshape(1, C1) - mean1 * a1

    out = pl.pallas_call(
        _k5_body,
        grid=(count // NT2,),
        in_specs=[
            pl.BlockSpec((NT2, C1), lambda i: (i, 0)),
            pl.BlockSpec((1, C1), lambda i: (0, 0)),
            pl.BlockSpec((1, C1), lambda i: (0, 0)),
        ],
        out_specs=pl.BlockSpec((NT2, C1), lambda i: (i, 0)),
        out_shape=jax.ShapeDtypeStruct((count, C1), jnp.float32),
    )(h2, a1, c1)

    return out.reshape(B, N, C1)


# speedup vs baseline: 30.3013x; 1.0834x over previous
import functools

import jax
import jax.numpy as jnp
from jax import lax
from jax.experimental import pallas as pl
from jax.experimental.pallas import tpu as pltpu
from jax.experimental.pallas import tpu_sc as plsc


NT1 = 512
NT2 = 512
NW = 32
PC = 128


def _k1_body(gbase, x1_ref, x2t_ref, w_ref, gi_ref):
    S = x2t_ref.shape[1]
    x1 = x1_ref[...]
    x2t = x2t_ref[...]
    cross = jax.lax.dot_general(
        x1, x2t, (((1,), (0,)), ((), ())),
        preferred_element_type=jnp.float32)
    x1c = [x1[:, c:c + 1] for c in range(3)]
    x2c = [x2t[c:c + 1, :] for c in range(3)]
    x1sq = x1c[0] * x1c[0] + x1c[1] * x1c[1] + x1c[2] * x1c[2]
    x2sq = x2c[0] * x2c[0] + x2c[1] * x2c[1] + x2c[2] * x2c[2]
    d = (-2.0) * cross + x1sq + x2sq

    iota_f = lax.broadcasted_iota(jnp.int32, (NT1, S), 1).astype(jnp.float32)
    ms, idxs = [], []
    dcur = d
    for k in range(3):
        m = jnp.min(dcur, axis=1, keepdims=True)
        hit = dcur == m
        idxf = jnp.min(jnp.where(hit, iota_f, jnp.float32(S)),
                       axis=1, keepdims=True)
        ms.append(m)
        idxs.append(idxf)
        if k < 2:
            dcur = jnp.where(iota_f == idxf, jnp.float32(jnp.inf), dcur)
    r = [1.0 / (m + 1e-8) for m in ms]
    norm = r[0] + r[1] + r[2]
    w_ref[...] = jnp.concatenate([rk / norm for rk in r], axis=1)
    gi_ref[...] = jnp.concatenate(idxs, axis=1).astype(jnp.int32) + gbase


def _sc_interp_body(p2_hbm, gidx_hbm, w_hbm, out_hbm,
                    idx_v, w_v, rows_v, out_v, sem):
    wid = lax.axis_index("s") * 2 + lax.axis_index("c")
    npts = out_hbm.shape[0]
    pw = npts // NW

    def chunk_body(ci, _):
        base = wid * pw + ci * PC
        ib = base * 3
        pltpu.sync_copy(gidx_hbm.at[pl.ds(ib, 3 * PC)], idx_v)
        pltpu.sync_copy(w_hbm.at[pl.ds(ib, 3 * PC)], w_v)
        cps = []
        for k in range(3):
            cps.append(pltpu.async_copy(
                p2_hbm.at[idx_v.at[pl.ds(128 * k, 128)]],
                rows_v.at[pl.ds(128 * k, 128)], sem))
        for cp in cps:
            cp.wait()

        def group_body(g, _):
            wvecs = [w_v[pl.ds(48 * g + 16 * j, 16)] for j in range(3)]
            p0 = 16 * g
            for i in range(16):
                w0 = wvecs[(3 * i) // 16][(3 * i) % 16]
                w1 = wvecs[(3 * i + 1) // 16][(3 * i + 1) % 16]
                w2 = wvecs[(3 * i + 2) // 16][(3 * i + 2) % 16]
                p = p0 + i
                for c in range(4):
                    sl = pl.ds(16 * c, 16)
                    out_v[p, sl] = (w0 * rows_v[3 * p, sl]
                                    + w1 * rows_v[3 * p + 1, sl]
                                    ) + w2 * rows_v[3 * p + 2, sl]
            return 0

        lax.fori_loop(0, PC // 16, group_body, 0)
        pltpu.sync_copy(out_v, out_hbm.at[pl.ds(base, PC)])
        return 0

    lax.fori_loop(0, pw // PC, chunk_body, 0)


def _k3_body(p1_ref, it_ref, w0a_ref, w0b_ref, b0_ref, h0_ref, s0_ref, q0_ref):
    h0 = jax.lax.dot_general(
        p1_ref[...], w0a_ref[...], (((1,), (0,)), ((), ())),
        preferred_element_type=jnp.float32)
    h0 = h0 + jax.lax.dot_general(
        it_ref[...], w0b_ref[...], (((1,), (0,)), ((), ())),
        preferred_element_type=jnp.float32)
    h0 = h0 + b0_ref[...]
    h0_ref[...] = h0

    @pl.when(pl.program_id(0) == 0)
    def _():
        s0_ref[...] = jnp.zeros_like(s0_ref)
        q0_ref[...] = jnp.zeros_like(q0_ref)

    s0_ref[...] += jnp.sum(h0, axis=0, keepdims=True)
    q0_ref[...] += jnp.sum(h0 * h0, axis=0, keepdims=True)


def _k4_body(h0_ref, a0_ref, c0_ref, w1t_ref, b1_ref,
             h2_ref, s1_ref, q1_ref):
    h1 = jnp.maximum(h0_ref[...] * a0_ref[...] + c0_ref[...], 0.0)
    h2 = jax.lax.dot_general(
        h1, w1t_ref[...], (((1,), (0,)), ((), ())),
        preferred_element_type=jnp.float32) + b1_ref[...]
    h2_ref[...] = h2

    @pl.when(pl.program_id(0) == 0)
    def _():
        s1_ref[...] = jnp.zeros_like(s1_ref)
        q1_ref[...] = jnp.zeros_like(q1_ref)

    s1_ref[...] += jnp.sum(h2, axis=0, keepdims=True)
    q1_ref[...] += jnp.sum(h2 * h2, axis=0, keepdims=True)


def _k5_body(h2_ref, a1_ref, c1_ref, out_ref):
    out_ref[...] = jnp.maximum(h2_ref[...] * a1_ref[...] + c1_ref[...], 0.0)


def kernel(xyz1, xyz2, points1, points2, W0, b0, gamma0, beta0,
           W1, b1, gamma1, beta1):
    B, N, _ = xyz1.shape
    S = xyz2.shape[1]
    D1 = points1.shape[2]
    D2 = points2.shape[2]
    C0 = W0.shape[0]
    C1 = W1.shape[0]
    nt = N // NT1
    count = B * N

    w0a = W0[:, :D1].T
    w0b = W0[:, D1:].T
    w1t = W1.T
    xyz2t = jnp.transpose(xyz2, (0, 2, 1))

    p2pad = jnp.concatenate(
        [points2.reshape(B * S, D2),
         jnp.zeros((B * S, 128 - D2), jnp.float32)], axis=1)

    sc_mesh = plsc.VectorSubcoreMesh(core_axis_name="c", subcore_axis_name="s")
    interp_parts = []
    for b in range(B):
        w3, gi3 = pl.pallas_call(
            functools.partial(_k1_body, b * S),
            grid=(nt,),
            in_specs=[
                pl.BlockSpec((NT1, 3), lambda i: (i, 0)),
                pl.BlockSpec((3, S), lambda i: (0, 0)),
            ],
            out_specs=[
                pl.BlockSpec((NT1, 3), lambda i: (i, 0)),
                pl.BlockSpec((NT1, 3), lambda i: (i, 0)),
            ],
            out_shape=[
                jax.ShapeDtypeStruct((N, 3), jnp.float32),
                jax.ShapeDtypeStruct((N, 3), jnp.int32),
            ],
        )(xyz1[b], xyz2t[b])
        interp_parts.append(pl.kernel(
            _sc_interp_body,
            mesh=sc_mesh,
            out_type=jax.ShapeDtypeStruct((N, D2), jnp.float32),
            scratch_types=[
                pltpu.VMEM((3 * PC,), jnp.int32),
                pltpu.VMEM((3 * PC,), jnp.float32),
                pltpu.VMEM((3 * PC, 128), jnp.float32),
                pltpu.VMEM((PC, D2), jnp.float32),
                pltpu.SemaphoreType.DMA,
            ],
        )(p2pad, gi3.reshape(N * 3), w3.reshape(N * 3)))
    interp = jnp.concatenate(interp_parts, axis=0)

    p1flat = points1.reshape(count, D1)
    h0f, s0, q0 = pl.pallas_call(
        _k3_body,
        grid=(count // NT2,),
        in_specs=[
            pl.BlockSpec((NT2, D1), lambda i: (i, 0)),
            pl.BlockSpec((NT2, D2), lambda i: (i, 0)),
            pl.BlockSpec((D1, C0), lambda i: (0, 0)),
            pl.BlockSpec((D2, C0), lambda i: (0, 0)),
            pl.BlockSpec((1, C0), lambda i: (0, 0)),
        ],
        out_specs=[
            pl.BlockSpec((NT2, C0), lambda i: (i, 0)),
            pl.BlockSpec((1, C0), lambda i: (0, 0)),
            pl.BlockSpec((1, C0), lambda i: (0, 0)),
        ],
        out_shape=[
            jax.ShapeDtypeStruct((count, C0), jnp.float32),
            jax.ShapeDtypeStruct((1, C0), jnp.float32),
            jax.ShapeDtypeStruct((1, C0), jnp.float32),
        ],
    )(p1flat, interp, w0a, w0b, b0.reshape(1, C0))

    mean0 = s0 / count
    var0 = q0 / count - mean0 * mean0
    a0 = gamma0.reshape(1, C0) / jnp.sqrt(var0 + 1e-5)
    c0 = beta0.reshape(1, C0) - mean0 * a0

    h2, s1, q1 = pl.pallas_call(
        _k4_body,
        grid=(count // NT2,),
        in_specs=[
            pl.BlockSpec((NT2, C0), lambda i: (i, 0)),
            pl.BlockSpec((1, C0), lambda i: (0, 0)),
            pl.BlockSpec((1, C0), lambda i: (0, 0)),
            pl.BlockSpec((C0, C1), lambda i: (0, 0)),
            pl.BlockSpec((1, C1), lambda i: (0, 0)),
        ],
        out_specs=[
            pl.BlockSpec((NT2, C1), lambda i: (i, 0)),
            pl.BlockSpec((1, C1), lambda i: (0, 0)),
            pl.BlockSpec((1, C1), lambda i: (0, 0)),
        ],
        out_shape=[
            jax.ShapeDtypeStruct((count, C1), jnp.float32),
            jax.ShapeDtypeStruct((1, C1), jnp.float32),
            jax.ShapeDtypeStruct((1, C1), jnp.float32),
        ],
    )(h0f, a0, c0, w1t, b1.reshape(1, C1))

    mean1 = s1 / count
    var1 = q1 / count - mean1 * mean1
    a1 = gamma1.reshape(1, C1) / jnp.sqrt(var1 + 1e-5)
    c1 = beta1.reshape(1, C1) - mean1 * a1

    out = pl.pallas_call(
        _k5_body,
        grid=(count // NT2,),
        in_specs=[
            pl.BlockSpec((NT2, C1), lambda i: (i, 0)),
            pl.BlockSpec((1, C1), lambda i: (0, 0)),
            pl.BlockSpec((1, C1), lambda i: (0, 0)),
        ],
        out_specs=pl.BlockSpec((NT2, C1), lambda i: (i, 0)),
        out_shape=jax.ShapeDtypeStruct((count, C1), jnp.float32),
    )(h2, a1, c1)

    return out.reshape(B, N, C1)


# speedup vs baseline: 36.0941x; 1.1912x over previous
import functools

import jax
import jax.numpy as jnp
from jax import lax
from jax.experimental import pallas as pl
from jax.experimental.pallas import tpu as pltpu
from jax.experimental.pallas import tpu_sc as plsc


NT1 = 512
NT2 = 2048
NW = 32
PC = 128


def _k1_body(gbase, x1_ref, x2t_ref, w_ref, gi_ref):
    S = x2t_ref.shape[1]
    x1 = x1_ref[...]
    x2t = x2t_ref[...]
    cross = jax.lax.dot_general(
        x1, x2t, (((1,), (0,)), ((), ())),
        preferred_element_type=jnp.float32)
    x1c = [x1[:, c:c + 1] for c in range(3)]
    x2c = [x2t[c:c + 1, :] for c in range(3)]
    x1sq = x1c[0] * x1c[0] + x1c[1] * x1c[1] + x1c[2] * x1c[2]
    x2sq = x2c[0] * x2c[0] + x2c[1] * x2c[1] + x2c[2] * x2c[2]
    d = (-2.0) * cross + x1sq + x2sq

    iota_f = lax.broadcasted_iota(jnp.int32, (NT1, S), 1).astype(jnp.float32)
    ms, idxs = [], []
    dcur = d
    for k in range(3):
        m = jnp.min(dcur, axis=1, keepdims=True)
        hit = dcur == m
        idxf = jnp.min(jnp.where(hit, iota_f, jnp.float32(S)),
                       axis=1, keepdims=True)
        ms.append(m)
        idxs.append(idxf)
        if k < 2:
            dcur = jnp.where(iota_f == idxf, jnp.float32(jnp.inf), dcur)
    r = [1.0 / (m + 1e-8) for m in ms]
    norm = r[0] + r[1] + r[2]
    w_ref[...] = jnp.concatenate([rk / norm for rk in r], axis=1)
    gi_ref[...] = jnp.concatenate(idxs, axis=1).astype(jnp.int32) + gbase


def _sc_interp_body(p2_hbm, gidx_hbm, w_hbm, out_hbm,
                    idx_v, w_v, rows_v, out_v, sem):
    wid = lax.axis_index("s") * 2 + lax.axis_index("c")
    npts = out_hbm.shape[0]
    pw = npts // NW

    def chunk_body(ci, _):
        base = wid * pw + ci * PC
        ib = base * 3
        pltpu.sync_copy(gidx_hbm.at[pl.ds(ib, 3 * PC)], idx_v)
        pltpu.sync_copy(w_hbm.at[pl.ds(ib, 3 * PC)], w_v)
        cps = []
        for k in range(3):
            cps.append(pltpu.async_copy(
                p2_hbm.at[idx_v.at[pl.ds(128 * k, 128)]],
                rows_v.at[pl.ds(128 * k, 128)], sem))
        for cp in cps:
            cp.wait()

        def group_body(g, _):
            wvecs = [w_v[pl.ds(48 * g + 16 * j, 16)] for j in range(3)]
            p0 = 16 * g
            for i in range(16):
                w0 = wvecs[(3 * i) // 16][(3 * i) % 16]
                w1 = wvecs[(3 * i + 1) // 16][(3 * i + 1) % 16]
                w2 = wvecs[(3 * i + 2) // 16][(3 * i + 2) % 16]
                p = p0 + i
                for c in range(4):
                    sl = pl.ds(16 * c, 16)
                    out_v[p, sl] = (w0 * rows_v[3 * p, sl]
                                    + w1 * rows_v[3 * p + 1, sl]
                                    ) + w2 * rows_v[3 * p + 2, sl]
            return 0

        lax.fori_loop(0, PC // 16, group_body, 0)
        pltpu.sync_copy(out_v, out_hbm.at[pl.ds(base, PC)])
        return 0

    lax.fori_loop(0, pw // PC, chunk_body, 0)


def _k3_body(p1_ref, it_ref, w0a_ref, w0b_ref, b0_ref, h0_ref, s0_ref, q0_ref):
    h0 = jax.lax.dot_general(
        p1_ref[...], w0a_ref[...], (((1,), (0,)), ((), ())),
        preferred_element_type=jnp.float32)
    h0 = h0 + jax.lax.dot_general(
        it_ref[...], w0b_ref[...], (((1,), (0,)), ((), ())),
        preferred_element_type=jnp.float32)
    h0 = h0 + b0_ref[...]
    h0_ref[...] = h0

    @pl.when(pl.program_id(0) == 0)
    def _():
        s0_ref[...] = jnp.zeros_like(s0_ref)
        q0_ref[...] = jnp.zeros_like(q0_ref)

    s0_ref[...] += jnp.sum(h0, axis=0, keepdims=True)
    q0_ref[...] += jnp.sum(h0 * h0, axis=0, keepdims=True)


def _k4_body(h0_ref, a0_ref, c0_ref, w1t_ref, b1_ref,
             h2_ref, s1_ref, q1_ref):
    h1 = jnp.maximum(h0_ref[...] * a0_ref[...] + c0_ref[...], 0.0)
    h2 = jax.lax.dot_general(
        h1, w1t_ref[...], (((1,), (0,)), ((), ())),
        preferred_element_type=jnp.float32) + b1_ref[...]
    h2_ref[...] = h2

    @pl.when(pl.program_id(0) == 0)
    def _():
        s1_ref[...] = jnp.zeros_like(s1_ref)
        q1_ref[...] = jnp.zeros_like(q1_ref)

    s1_ref[...] += jnp.sum(h2, axis=0, keepdims=True)
    q1_ref[...] += jnp.sum(h2 * h2, axis=0, keepdims=True)


def _k5_body(h2_ref, a1_ref, c1_ref, out_ref):
    out_ref[...] = jnp.maximum(h2_ref[...] * a1_ref[...] + c1_ref[...], 0.0)


def kernel(xyz1, xyz2, points1, points2, W0, b0, gamma0, beta0,
           W1, b1, gamma1, beta1):
    B, N, _ = xyz1.shape
    S = xyz2.shape[1]
    D1 = points1.shape[2]
    D2 = points2.shape[2]
    C0 = W0.shape[0]
    C1 = W1.shape[0]
    nt = N // NT1
    count = B * N

    w0a = W0[:, :D1].T
    w0b = W0[:, D1:].T
    w1t = W1.T
    xyz2t = jnp.transpose(xyz2, (0, 2, 1))

    p2pad = jnp.concatenate(
        [points2.reshape(B * S, D2),
         jnp.zeros((B * S, 128 - D2), jnp.float32)], axis=1)

    sc_mesh = plsc.VectorSubcoreMesh(core_axis_name="c", subcore_axis_name="s")
    interp_parts = []
    for b in range(B):
        w3, gi3 = pl.pallas_call(
            functools.partial(_k1_body, b * S),
            grid=(nt,),
            in_specs=[
                pl.BlockSpec((NT1, 3), lambda i: (i, 0)),
                pl.BlockSpec((3, S), lambda i: (0, 0)),
            ],
            out_specs=[
                pl.BlockSpec((NT1, 3), lambda i: (i, 0)),
                pl.BlockSpec((NT1, 3), lambda i: (i, 0)),
            ],
            out_shape=[
                jax.ShapeDtypeStruct((N, 3), jnp.float32),
                jax.ShapeDtypeStruct((N, 3), jnp.int32),
            ],
        )(xyz1[b], xyz2t[b])
        interp_parts.append(pl.kernel(
            _sc_interp_body,
            mesh=sc_mesh,
            out_type=jax.ShapeDtypeStruct((N, D2), jnp.float32),
            scratch_types=[
                pltpu.VMEM((3 * PC,), jnp.int32),
                pltpu.VMEM((3 * PC,), jnp.float32),
                pltpu.VMEM((3 * PC, 128), jnp.float32),
                pltpu.VMEM((PC, D2), jnp.float32),
                pltpu.SemaphoreType.DMA,
            ],
        )(p2pad, gi3.reshape(N * 3), w3.reshape(N * 3)))
    interp = jnp.concatenate(interp_parts, axis=0)

    p1flat = points1.reshape(count, D1)
    h0f, s0, q0 = pl.pallas_call(
        _k3_body,
        grid=(count // NT2,),
        in_specs=[
            pl.BlockSpec((NT2, D1), lambda i: (i, 0)),
            pl.BlockSpec((NT2, D2), lambda i: (i, 0)),
            pl.BlockSpec((D1, C0), lambda i: (0, 0)),
            pl.BlockSpec((D2, C0), lambda i: (0, 0)),
            pl.BlockSpec((1, C0), lambda i: (0, 0)),
        ],
        out_specs=[
            pl.BlockSpec((NT2, C0), lambda i: (i, 0)),
            pl.BlockSpec((1, C0), lambda i: (0, 0)),
            pl.BlockSpec((1, C0), lambda i: (0, 0)),
        ],
        out_shape=[
            jax.ShapeDtypeStruct((count, C0), jnp.float32),
            jax.ShapeDtypeStruct((1, C0), jnp.float32),
            jax.ShapeDtypeStruct((1, C0), jnp.float32),
        ],
    )(p1flat, interp, w0a, w0b, b0.reshape(1, C0))

    mean0 = s0 / count
    var0 = q0 / count - mean0 * mean0
    a0 = gamma0.reshape(1, C0) / jnp.sqrt(var0 + 1e-5)
    c0 = beta0.reshape(1, C0) - mean0 * a0

    h2, s1, q1 = pl.pallas_call(
        _k4_body,
        grid=(count // NT2,),
        in_specs=[
            pl.BlockSpec((NT2, C0), lambda i: (i, 0)),
            pl.BlockSpec((1, C0), lambda i: (0, 0)),
            pl.BlockSpec((1, C0), lambda i: (0, 0)),
            pl.BlockSpec((C0, C1), lambda i: (0, 0)),
            pl.BlockSpec((1, C1), lambda i: (0, 0)),
        ],
        out_specs=[
            pl.BlockSpec((NT2, C1), lambda i: (i, 0)),
            pl.BlockSpec((1, C1), lambda i: (0, 0)),
            pl.BlockSpec((1, C1), lambda i: (0, 0)),
        ],
        out_shape=[
            jax.ShapeDtypeStruct((count, C1), jnp.float32),
            jax.ShapeDtypeStruct((1, C1), jnp.float32),
            jax.ShapeDtypeStruct((1, C1), jnp.float32),
        ],
    )(h0f, a0, c0, w1t, b1.reshape(1, C1))

    mean1 = s1 / count
    var1 = q1 / count - mean1 * mean1
    a1 = gamma1.reshape(1, C1) / jnp.sqrt(var1 + 1e-5)
    c1 = beta1.reshape(1, C1) - mean1 * a1

    out = pl.pallas_call(
        _k5_body,
        grid=(count // NT2,),
        in_specs=[
            pl.BlockSpec((NT2, C1), lambda i: (i, 0)),
            pl.BlockSpec((1, C1), lambda i: (0, 0)),
            pl.BlockSpec((1, C1), lambda i: (0, 0)),
        ],
        out_specs=pl.BlockSpec((NT2, C1), lambda i: (i, 0)),
        out_shape=jax.ShapeDtypeStruct((count, C1), jnp.float32),
    )(h2, a1, c1)

    return out.reshape(B, N, C1)


# speedup vs baseline: 36.2680x; 1.0048x over previous
import functools

import jax
import jax.numpy as jnp
from jax import lax
from jax.experimental import pallas as pl
from jax.experimental.pallas import tpu as pltpu
from jax.experimental.pallas import tpu_sc as plsc


NT1 = 1024
NT2 = 2048
NW = 32
PC = 128


def _k1_body(gbase, x1_ref, x2t_ref, w_ref, gi_ref):
    S = x2t_ref.shape[1]
    x1 = x1_ref[...]
    x2t = x2t_ref[...]
    cross = jax.lax.dot_general(
        x1, x2t, (((1,), (0,)), ((), ())),
        preferred_element_type=jnp.float32)
    x1c = [x1[:, c:c + 1] for c in range(3)]
    x2c = [x2t[c:c + 1, :] for c in range(3)]
    x1sq = x1c[0] * x1c[0] + x1c[1] * x1c[1] + x1c[2] * x1c[2]
    x2sq = x2c[0] * x2c[0] + x2c[1] * x2c[1] + x2c[2] * x2c[2]
    d = (-2.0) * cross + x1sq + x2sq

    iota_f = lax.broadcasted_iota(jnp.int32, (NT1, S), 1).astype(jnp.float32)
    ms, idxs = [], []
    dcur = d
    for k in range(3):
        m = jnp.min(dcur, axis=1, keepdims=True)
        hit = dcur == m
        idxf = jnp.min(jnp.where(hit, iota_f, jnp.float32(S)),
                       axis=1, keepdims=True)
        ms.append(m)
        idxs.append(idxf)
        if k < 2:
            dcur = jnp.where(iota_f == idxf, jnp.float32(jnp.inf), dcur)
    r = [1.0 / (m + 1e-8) for m in ms]
    norm = r[0] + r[1] + r[2]
    w_ref[...] = jnp.concatenate([rk / norm for rk in r], axis=1)
    gi_ref[...] = jnp.concatenate(idxs, axis=1).astype(jnp.int32) + gbase


def _sc_interp_body(p2_hbm, gidx_hbm, w_hbm, out_hbm,
                    idx_v, w_v, rows_v, out_v, sem):
    wid = lax.axis_index("s") * 2 + lax.axis_index("c")
    npts = out_hbm.shape[0]
    pw = npts // NW

    def chunk_body(ci, _):
        base = wid * pw + ci * PC
        ib = base * 3
        pltpu.sync_copy(gidx_hbm.at[pl.ds(ib, 3 * PC)], idx_v)
        pltpu.sync_copy(w_hbm.at[pl.ds(ib, 3 * PC)], w_v)
        cps = []
        for k in range(3):
            cps.append(pltpu.async_copy(
                p2_hbm.at[idx_v.at[pl.ds(128 * k, 128)]],
                rows_v.at[pl.ds(128 * k, 128)], sem))
        for cp in cps:
            cp.wait()

        def group_body(g, _):
            wvecs = [w_v[pl.ds(48 * g + 16 * j, 16)] for j in range(3)]
            p0 = 16 * g
            for i in range(16):
                w0 = wvecs[(3 * i) // 16][(3 * i) % 16]
                w1 = wvecs[(3 * i + 1) // 16][(3 * i + 1) % 16]
                w2 = wvecs[(3 * i + 2) // 16][(3 * i + 2) % 16]
                p = p0 + i
                for c in range(4):
                    sl = pl.ds(16 * c, 16)
                    out_v[p, sl] = (w0 * rows_v[3 * p, sl]
                                    + w1 * rows_v[3 * p + 1, sl]
                                    ) + w2 * rows_v[3 * p + 2, sl]
            return 0

        lax.fori_loop(0, PC // 16, group_body, 0)
        pltpu.sync_copy(out_v, out_hbm.at[pl.ds(base, PC)])
        return 0

    lax.fori_loop(0, pw // PC, chunk_body, 0)


def _k3_body(p1_ref, it_ref, w0a_ref, w0b_ref, b0_ref, h0_ref, s0_ref, q0_ref):
    h0 = jax.lax.dot_general(
        p1_ref[...], w0a_ref[...], (((1,), (0,)), ((), ())),
        preferred_element_type=jnp.float32)
    h0 = h0 + jax.lax.dot_general(
        it_ref[...], w0b_ref[...], (((1,), (0,)), ((), ())),
        preferred_element_type=jnp.float32)
    h0 = h0 + b0_ref[...]
    h0_ref[...] = h0

    @pl.when(pl.program_id(0) == 0)
    def _():
        s0_ref[...] = jnp.zeros_like(s0_ref)
        q0_ref[...] = jnp.zeros_like(q0_ref)

    s0_ref[...] += jnp.sum(h0, axis=0, keepdims=True)
    q0_ref[...] += jnp.sum(h0 * h0, axis=0, keepdims=True)


def _k4_body(h0_ref, a0_ref, c0_ref, w1t_ref, b1_ref,
             h2_ref, s1_ref, q1_ref):
    h1 = jnp.maximum(h0_ref[...] * a0_ref[...] + c0_ref[...], 0.0)
    h2 = jax.lax.dot_general(
        h1, w1t_ref[...], (((1,), (0,)), ((), ())),
        preferred_element_type=jnp.float32) + b1_ref[...]
    h2_ref[...] = h2

    @pl.when(pl.program_id(0) == 0)
    def _():
        s1_ref[...] = jnp.zeros_like(s1_ref)
        q1_ref[...] = jnp.zeros_like(q1_ref)

    s1_ref[...] += jnp.sum(h2, axis=0, keepdims=True)
    q1_ref[...] += jnp.sum(h2 * h2, axis=0, keepdims=True)


def _k5_body(h2_ref, a1_ref, c1_ref, out_ref):
    out_ref[...] = jnp.maximum(h2_ref[...] * a1_ref[...] + c1_ref[...], 0.0)


def kernel(xyz1, xyz2, points1, points2, W0, b0, gamma0, beta0,
           W1, b1, gamma1, beta1):
    B, N, _ = xyz1.shape
    S = xyz2.shape[1]
    D1 = points1.shape[2]
    D2 = points2.shape[2]
    C0 = W0.shape[0]
    C1 = W1.shape[0]
    nt = N // NT1
    count = B * N

    w0a = W0[:, :D1].T
    w0b = W0[:, D1:].T
    w1t = W1.T
    xyz2t = jnp.transpose(xyz2, (0, 2, 1))

    p2pad = jnp.concatenate(
        [points2.reshape(B * S, D2),
         jnp.zeros((B * S, 128 - D2), jnp.float32)], axis=1)

    sc_mesh = plsc.VectorSubcoreMesh(core_axis_name="c", subcore_axis_name="s")
    interp_parts = []
    for b in range(B):
        w3, gi3 = pl.pallas_call(
            functools.partial(_k1_body, b * S),
            grid=(nt,),
            in_specs=[
                pl.BlockSpec((NT1, 3), lambda i: (i, 0)),
                pl.BlockSpec((3, S), lambda i: (0, 0)),
            ],
            out_specs=[
                pl.BlockSpec((NT1, 3), lambda i: (i, 0)),
                pl.BlockSpec((NT1, 3), lambda i: (i, 0)),
            ],
            out_shape=[
                jax.ShapeDtypeStruct((N, 3), jnp.float32),
                jax.ShapeDtypeStruct((N, 3), jnp.int32),
            ],
        )(xyz1[b], xyz2t[b])
        interp_parts.append(pl.kernel(
            _sc_interp_body,
            mesh=sc_mesh,
            out_type=jax.ShapeDtypeStruct((N, D2), jnp.float32),
            scratch_types=[
                pltpu.VMEM((3 * PC,), jnp.int32),
                pltpu.VMEM((3 * PC,), jnp.float32),
                pltpu.VMEM((3 * PC, 128), jnp.float32),
                pltpu.VMEM((PC, D2), jnp.float32),
                pltpu.SemaphoreType.DMA,
            ],
        )(p2pad, gi3.reshape(N * 3), w3.reshape(N * 3)))
    interp = jnp.concatenate(interp_parts, axis=0)

    p1flat = points1.reshape(count, D1)
    h0f, s0, q0 = pl.pallas_call(
        _k3_body,
        grid=(count // NT2,),
        in_specs=[
            pl.BlockSpec((NT2, D1), lambda i: (i, 0)),
            pl.BlockSpec((NT2, D2), lambda i: (i, 0)),
            pl.BlockSpec((D1, C0), lambda i: (0, 0)),
            pl.BlockSpec((D2, C0), lambda i: (0, 0)),
            pl.BlockSpec((1, C0), lambda i: (0, 0)),
        ],
        out_specs=[
            pl.BlockSpec((NT2, C0), lambda i: (i, 0)),
            pl.BlockSpec((1, C0), lambda i: (0, 0)),
            pl.BlockSpec((1, C0), lambda i: (0, 0)),
        ],
        out_shape=[
            jax.ShapeDtypeStruct((count, C0), jnp.float32),
            jax.ShapeDtypeStruct((1, C0), jnp.float32),
            jax.ShapeDtypeStruct((1, C0), jnp.float32),
        ],
    )(p1flat, interp, w0a, w0b, b0.reshape(1, C0))

    mean0 = s0 / count
    var0 = q0 / count - mean0 * mean0
    a0 = gamma0.reshape(1, C0) / jnp.sqrt(var0 + 1e-5)
    c0 = beta0.reshape(1, C0) - mean0 * a0

    h2, s1, q1 = pl.pallas_call(
        _k4_body,
        grid=(count // NT2,),
        in_specs=[
            pl.BlockSpec((NT2, C0), lambda i: (i, 0)),
            pl.BlockSpec((1, C0), lambda i: (0, 0)),
            pl.BlockSpec((1, C0), lambda i: (0, 0)),
            pl.BlockSpec((C0, C1), lambda i: (0, 0)),
            pl.BlockSpec((1, C1), lambda i: (0, 0)),
        ],
        out_specs=[
            pl.BlockSpec((NT2, C1), lambda i: (i, 0)),
            pl.BlockSpec((1, C1), lambda i: (0, 0)),
            pl.BlockSpec((1, C1), lambda i: (0, 0)),
        ],
        out_shape=[
            jax.ShapeDtypeStruct((count, C1), jnp.float32),
            jax.ShapeDtypeStruct((1, C1), jnp.float32),
            jax.ShapeDtypeStruct((1, C1), jnp.float32),
        ],
    )(h0f, a0, c0, w1t, b1.reshape(1, C1))

    mean1 = s1 / count
    var1 = q1 / count - mean1 * mean1
    a1 = gamma1.reshape(1, C1) / jnp.sqrt(var1 + 1e-5)
    c1 = beta1.reshape(1, C1) - mean1 * a1

    out = pl.pallas_call(
        _k5_body,
        grid=(count // NT2,),
        in_specs=[
            pl.BlockSpec((NT2, C1), lambda i: (i, 0)),
            pl.BlockSpec((1, C1), lambda i: (0, 0)),
            pl.BlockSpec((1, C1), lambda i: (0, 0)),
        ],
        out_specs=pl.BlockSpec((NT2, C1), lambda i: (i, 0)),
        out_shape=jax.ShapeDtypeStruct((count, C1), jnp.float32),
    )(h2, a1, c1)

    return out.reshape(B, N, C1)
